# Initial kernel scaffold; baseline (speedup 1.0000x reference)
import jax
import jax.numpy as jnp
from jax.experimental import pallas as pl


def kernel(inputs, edge_index, W, b):
    raise NotImplementedError("write your pallas kernel here")


# speedup vs baseline: 3.2267x; 3.2267x over previous
import functools

import jax
import jax.numpy as jnp
from jax import lax
from jax.experimental import pallas as pl
from jax.experimental.pallas import tpu as pltpu
from jax.experimental.pallas import tpu_sc as plsc

_N_NODES = 10000

NC = 2
NS = 16
NW = NC * NS
LANES = 16
CHUNK = 80


def _mesh():
    return plsc.VectorSubcoreMesh(core_axis_name="c", subcore_axis_name="s")


def _zero_rows(rows_v, d):
    for r in range(CHUNK):
        for j in range(d // LANES):
            rows_v[r, pl.ds(j * LANES, LANES)] = jnp.zeros((LANES,), jnp.float32)


def _fill_iota(idx_v, start):
    for j in range(CHUNK // LANES):
        idx_v[pl.ds(j * LANES, LANES)] = (
            lax.iota(jnp.int32, LANES) + start + j * LANES)


def _seg_sum_call(n_nodes, n_edges, d):
    ew = n_edges // NW
    assert ew * NW == n_edges and ew % CHUNK == 0
    n_chunks = ew // CHUNK
    rpt = n_nodes // NS

    def body(inputs_hbm, dst_hbm, ns_out, deg_out,
             idx_v, rows_v, ns_sh, sem):
        c = lax.axis_index("c")
        s = lax.axis_index("s")
        wid = s * NC + c
        r0 = s * rpt
        base = wid * ew

        _zero_rows(rows_v, d)
        for k in range(rpt // CHUNK):
            _fill_iota(idx_v, r0 + k * CHUNK)
            pltpu.sync_copy(rows_v, ns_sh.at[idx_v])
        plsc.subcore_barrier()

        def cbody(i, carry):
            off = base + i * CHUNK
            pltpu.sync_copy(dst_hbm.at[pl.ds(off, CHUNK)], idx_v)
            pltpu.sync_copy(inputs_hbm.at[pl.ds(off, CHUNK)], rows_v)
            pltpu.sync_copy(rows_v, ns_sh.at[idx_v], add=True)
            return carry
        lax.fori_loop(0, n_chunks, cbody, 0)
        plsc.subcore_barrier()

        for k in range(rpt // CHUNK):
            rk = r0 + k * CHUNK
            _fill_iota(idx_v, rk)
            pltpu.async_copy(ns_sh.at[idx_v], rows_v, sem).wait()
            pltpu.sync_copy(rows_v, ns_out.at[c, pl.ds(rk, CHUNK)])

        _zero_rows(rows_v, d)
        for k in range(rpt // CHUNK):
            _fill_iota(idx_v, r0 + k * CHUNK)
            pltpu.sync_copy(rows_v, ns_sh.at[idx_v])
        for r in range(CHUNK):
            for j in range(d // LANES):
                rows_v[r, pl.ds(j * LANES, LANES)] = jnp.ones((LANES,), jnp.float32)
        plsc.subcore_barrier()

        def dbody(i, carry):
            off = base + i * CHUNK
            pltpu.sync_copy(dst_hbm.at[pl.ds(off, CHUNK)], idx_v)
            pltpu.sync_copy(rows_v, ns_sh.at[idx_v], add=True)
            return carry
        lax.fori_loop(0, n_chunks, dbody, 0)
        plsc.subcore_barrier()

        for k in range(rpt // CHUNK):
            rk = r0 + k * CHUNK
            _fill_iota(idx_v, rk)
            pltpu.async_copy(ns_sh.at[idx_v], rows_v, sem).wait()
            pltpu.sync_copy(rows_v, deg_out.at[c, pl.ds(rk, CHUNK)])

    return pl.kernel(
        body,
        out_type=(
            jax.ShapeDtypeStruct((NC, n_nodes, d), jnp.float32),
            jax.ShapeDtypeStruct((NC, n_nodes, d), jnp.float32),
        ),
        mesh=_mesh(),
        scratch_types=[
            pltpu.VMEM((CHUNK,), jnp.int32),
            pltpu.VMEM((CHUNK, d), jnp.float32),
            pltpu.VMEM_SHARED((n_nodes, d), jnp.float32),
            pltpu.SemaphoreType.DMA,
        ],
    )


def _gather_scatter_call(n_nodes, n_edges, d):
    ew = n_edges // NW
    n_chunks = ew // CHUNK
    rpt = n_nodes // NS

    def body(nodeh_hbm, src_hbm, dst_hbm, h_out,
             sidx_v, didx_v, rows_v, h_sh, sem):
        c = lax.axis_index("c")
        s = lax.axis_index("s")
        wid = s * NC + c
        r0 = s * rpt

        _zero_rows(rows_v, d)
        for k in range(rpt // CHUNK):
            _fill_iota(didx_v, r0 + k * CHUNK)
            pltpu.sync_copy(rows_v, h_sh.at[didx_v])
        plsc.subcore_barrier()

        base = wid * ew

        def cbody(i, carry):
            off = base + i * CHUNK
            pltpu.sync_copy(src_hbm.at[pl.ds(off, CHUNK)], sidx_v)
            pltpu.async_copy(nodeh_hbm.at[sidx_v], rows_v, sem).wait()
            pltpu.sync_copy(dst_hbm.at[pl.ds(off, CHUNK)], didx_v)
            pltpu.sync_copy(rows_v, h_sh.at[didx_v], add=True)
            return carry
        lax.fori_loop(0, n_chunks, cbody, 0)
        plsc.subcore_barrier()

        for k in range(rpt // CHUNK):
            rk = r0 + k * CHUNK
            _fill_iota(didx_v, rk)
            pltpu.async_copy(h_sh.at[didx_v], rows_v, sem).wait()
            pltpu.sync_copy(rows_v, h_out.at[c, pl.ds(rk, CHUNK)])

    return pl.kernel(
        body,
        out_type=jax.ShapeDtypeStruct((NC, n_nodes, d), jnp.float32),
        mesh=_mesh(),
        scratch_types=[
            pltpu.VMEM((CHUNK,), jnp.int32),
            pltpu.VMEM((CHUNK,), jnp.int32),
            pltpu.VMEM((CHUNK, d), jnp.float32),
            pltpu.VMEM_SHARED((n_nodes, d), jnp.float32),
            pltpu.SemaphoreType.DMA,
        ],
    )


def _edge_out_call(n_nodes, n_edges, d):
    ew = n_edges // NW
    n_chunks = ew // CHUNK

    def body(p_hbm, src_hbm, dst_hbm, out_hbm, sidx_v, didx_v, a_v, b_v, sem):
        c = lax.axis_index("c")
        s = lax.axis_index("s")
        wid = s * NC + c
        base = wid * ew

        def cbody(i, carry):
            off = base + i * CHUNK
            pltpu.sync_copy(src_hbm.at[pl.ds(off, CHUNK)], sidx_v)
            pltpu.async_copy(p_hbm.at[sidx_v], a_v, sem).wait()
            pltpu.sync_copy(dst_hbm.at[pl.ds(off, CHUNK)], didx_v)
            pltpu.async_copy(p_hbm.at[didx_v], b_v, sem).wait()

            def rbody(r, rc):
                for j in range(d // LANES):
                    sl = pl.ds(j * LANES, LANES)
                    a_v[r, sl] = a_v[r, sl] + b_v[r, sl]
                return rc
            lax.fori_loop(0, CHUNK, rbody, 0)
            pltpu.sync_copy(a_v, out_hbm.at[pl.ds(off, CHUNK)])
            return carry
        lax.fori_loop(0, n_chunks, cbody, 0)

    return pl.kernel(
        body,
        out_type=jax.ShapeDtypeStruct((n_edges, d), jnp.float32),
        mesh=_mesh(),
        scratch_types=[
            pltpu.VMEM((CHUNK,), jnp.int32),
            pltpu.VMEM((CHUNK,), jnp.int32),
            pltpu.VMEM((CHUNK, d), jnp.float32),
            pltpu.VMEM((CHUNK, d), jnp.float32),
            pltpu.SemaphoreType.DMA,
        ],
    )


def _combine_body(ns_ref, deg_ref, out_ref):
    ns = ns_ref[0] + ns_ref[1]
    deg = deg_ref[0, :, 0] + deg_ref[1, :, 0]
    out_ref[...] = ns / jnp.maximum(deg, 1.0)[:, None]


def _linear_body(h_ref, w_ref, b_ref, out_ref):
    h = h_ref[0] + h_ref[1]
    p = lax.dot_general(h, w_ref[...], (((1,), (1,)), ((), ())),
                        preferred_element_type=jnp.float32)
    out_ref[...] = 0.5 * p + 0.5 * b_ref[...]


def kernel(inputs, edge_index, W, b):
    n_edges, d = inputs.shape
    n_nodes = ((_N_NODES + NS * CHUNK - 1) // (NS * CHUNK)) * (NS * CHUNK)
    src = edge_index[0]
    dst = edge_index[1]

    ns_p, deg_p = _seg_sum_call(n_nodes, n_edges, d)(inputs, dst)

    node_h = pl.pallas_call(
        _combine_body,
        out_shape=jax.ShapeDtypeStruct((n_nodes, d), jnp.float32),
    )(ns_p, deg_p)

    h_p = _gather_scatter_call(n_nodes, n_edges, d)(node_h, src, dst)

    p = pl.pallas_call(
        _linear_body,
        out_shape=jax.ShapeDtypeStruct((n_nodes, d), jnp.float32),
    )(h_p, W, b.reshape(1, d))

    return _edge_out_call(n_nodes, n_edges, d)(p, src, dst)


# speedup vs baseline: 4.2937x; 1.3307x over previous
import functools

import jax
import jax.numpy as jnp
from jax import lax
from jax.experimental import pallas as pl
from jax.experimental.pallas import tpu as pltpu
from jax.experimental.pallas import tpu_sc as plsc

_N_NODES = 10000

NC = 2
NS = 16
NW = NC * NS
LANES = 16
CHUNK = 80


def _mesh():
    return plsc.VectorSubcoreMesh(core_axis_name="c", subcore_axis_name="s")


def _zero_rows(rows_v, d):
    for r in range(CHUNK):
        for j in range(d // LANES):
            rows_v[r, pl.ds(j * LANES, LANES)] = jnp.zeros((LANES,), jnp.float32)


def _fill_iota(idx_v, start):
    for j in range(CHUNK // LANES):
        idx_v[pl.ds(j * LANES, LANES)] = (
            lax.iota(jnp.int32, LANES) + start + j * LANES)


def _seg_sum_call(n_nodes, n_edges, d):
    ew = n_edges // NW
    assert ew * NW == n_edges and ew % CHUNK == 0
    n_chunks = ew // CHUNK
    rpt = n_nodes // NS

    def body(inputs_hbm, dst_hbm, ns_out, deg_out,
             idx_v, rows_v, ns_sh, sem):
        c = lax.axis_index("c")
        s = lax.axis_index("s")
        wid = s * NC + c
        r0 = s * rpt
        base = wid * ew

        _zero_rows(rows_v, d)
        for k in range(rpt // CHUNK):
            _fill_iota(idx_v, r0 + k * CHUNK)
            pltpu.sync_copy(rows_v, ns_sh.at[idx_v])
        plsc.subcore_barrier()

        def cbody(i, carry):
            off = base + i * CHUNK
            pltpu.sync_copy(dst_hbm.at[pl.ds(off, CHUNK)], idx_v)
            pltpu.sync_copy(inputs_hbm.at[pl.ds(off, CHUNK)], rows_v)
            pltpu.sync_copy(rows_v, ns_sh.at[idx_v], add=True)
            return carry
        lax.fori_loop(0, n_chunks, cbody, 0)
        plsc.subcore_barrier()

        for k in range(rpt // CHUNK):
            rk = r0 + k * CHUNK
            _fill_iota(idx_v, rk)
            pltpu.async_copy(ns_sh.at[idx_v], rows_v, sem).wait()
            pltpu.sync_copy(rows_v, ns_out.at[c, pl.ds(rk, CHUNK)])

        _zero_rows(rows_v, d)
        for k in range(rpt // CHUNK):
            _fill_iota(idx_v, r0 + k * CHUNK)
            pltpu.sync_copy(rows_v, ns_sh.at[idx_v])
        for r in range(CHUNK):
            for j in range(d // LANES):
                rows_v[r, pl.ds(j * LANES, LANES)] = jnp.ones((LANES,), jnp.float32)
        plsc.subcore_barrier()

        def dbody(i, carry):
            off = base + i * CHUNK
            pltpu.sync_copy(dst_hbm.at[pl.ds(off, CHUNK)], idx_v)
            pltpu.sync_copy(rows_v, ns_sh.at[idx_v], add=True)
            return carry
        lax.fori_loop(0, n_chunks, dbody, 0)
        plsc.subcore_barrier()

        for k in range(rpt // CHUNK):
            rk = r0 + k * CHUNK
            _fill_iota(idx_v, rk)
            pltpu.async_copy(ns_sh.at[idx_v], rows_v, sem).wait()
            pltpu.sync_copy(rows_v, deg_out.at[c, pl.ds(rk, CHUNK)])

    return pl.kernel(
        body,
        out_type=(
            jax.ShapeDtypeStruct((NC, n_nodes, d), jnp.float32),
            jax.ShapeDtypeStruct((NC, n_nodes, d), jnp.float32),
        ),
        mesh=_mesh(),
        scratch_types=[
            pltpu.VMEM((CHUNK,), jnp.int32),
            pltpu.VMEM((CHUNK, d), jnp.float32),
            pltpu.VMEM_SHARED((n_nodes, d), jnp.float32),
            pltpu.SemaphoreType.DMA,
        ],
    )


def _gather_scatter_call(n_nodes, n_edges, d):
    ew = n_edges // NW
    n_chunks = ew // CHUNK
    rpt = n_nodes // NS

    def body(nodeh_hbm, src_hbm, dst_hbm, h_out,
             sidx_v, didx_v, rows_v, h_sh, sem):
        c = lax.axis_index("c")
        s = lax.axis_index("s")
        wid = s * NC + c
        r0 = s * rpt

        _zero_rows(rows_v, d)
        for k in range(rpt // CHUNK):
            _fill_iota(didx_v, r0 + k * CHUNK)
            pltpu.sync_copy(rows_v, h_sh.at[didx_v])
        plsc.subcore_barrier()

        base = wid * ew

        def cbody(i, carry):
            off = base + i * CHUNK
            pltpu.sync_copy(src_hbm.at[pl.ds(off, CHUNK)], sidx_v)
            pltpu.async_copy(nodeh_hbm.at[sidx_v], rows_v, sem).wait()
            pltpu.sync_copy(dst_hbm.at[pl.ds(off, CHUNK)], didx_v)
            pltpu.sync_copy(rows_v, h_sh.at[didx_v], add=True)
            return carry
        lax.fori_loop(0, n_chunks, cbody, 0)
        plsc.subcore_barrier()

        for k in range(rpt // CHUNK):
            rk = r0 + k * CHUNK
            _fill_iota(didx_v, rk)
            pltpu.async_copy(h_sh.at[didx_v], rows_v, sem).wait()
            pltpu.sync_copy(rows_v, h_out.at[c, pl.ds(rk, CHUNK)])

    return pl.kernel(
        body,
        out_type=jax.ShapeDtypeStruct((NC, n_nodes, d), jnp.float32),
        mesh=_mesh(),
        scratch_types=[
            pltpu.VMEM((CHUNK,), jnp.int32),
            pltpu.VMEM((CHUNK,), jnp.int32),
            pltpu.VMEM((CHUNK, d), jnp.float32),
            pltpu.VMEM_SHARED((n_nodes, d), jnp.float32),
            pltpu.SemaphoreType.DMA,
        ],
    )


def _edge_out_call(n_nodes, n_edges, d):
    ew = n_edges // NW
    n_chunks = ew // CHUNK
    assert n_chunks % 2 == 1

    def body(p_hbm, src_hbm, dst_hbm, out_hbm,
             si0, si1, di0, di1, av0, av1, bv0, bv1,
             gsem0, gsem1, isem0, isem1):
        c = lax.axis_index("c")
        s = lax.axis_index("s")
        wid = s * NC + c
        base = wid * ew
        limit = base + ew
        si = (si0, si1)
        di = (di0, di1)
        av = (av0, av1)
        bv = (bv0, bv1)
        gsem = (gsem0, gsem1)
        isem = (isem0, isem1)

        def start_idx(j, slot):
            off = base + j * CHUNK
            pltpu.async_copy(src_hbm.at[pl.ds(off, CHUNK)], si[slot], isem[slot])
            pltpu.async_copy(dst_hbm.at[pl.ds(off, CHUNK)], di[slot], isem[slot])

        def wait_idx(slot):
            pltpu.make_async_copy(src_hbm.at[pl.ds(base, CHUNK)], si[slot],
                                  isem[slot]).wait()
            pltpu.make_async_copy(dst_hbm.at[pl.ds(base, CHUNK)], di[slot],
                                  isem[slot]).wait()

        def start_gathers(slot):
            pltpu.async_copy(p_hbm.at[si[slot]], av[slot], gsem[slot])
            pltpu.async_copy(p_hbm.at[di[slot]], bv[slot], gsem[slot])

        def wait_gathers(slot):
            pltpu.make_async_copy(p_hbm.at[pl.ds(0, CHUNK)], av[slot],
                                  gsem[slot]).wait()
            pltpu.make_async_copy(p_hbm.at[pl.ds(0, CHUNK)], bv[slot],
                                  gsem[slot]).wait()

        def add_and_store(j, slot):
            def rbody(rr, rc):
                for q in range(8):
                    r = rr * 8 + q
                    for jj in range(d // LANES):
                        sl = pl.ds(jj * LANES, LANES)
                        av[slot][r, sl] = av[slot][r, sl] + bv[slot][r, sl]
                return rc
            lax.fori_loop(0, CHUNK // 8, rbody, 0)
            pltpu.sync_copy(av[slot], out_hbm.at[pl.ds(base + j * CHUNK, CHUNK)])

        start_idx(0, 0)
        wait_idx(0)
        start_gathers(0)
        start_idx(1, 1)

        def pbody(ip, carry):
            for b2 in range(2):
                j = ip * 2 + b2
                slot = b2
                other = 1 - b2
                wait_gathers(slot)
                wait_idx(other)
                start_gathers(other)

                @pl.when(base + (j + 2) * CHUNK < limit)
                def _():
                    start_idx(j + 2, slot)
                add_and_store(j, slot)
            return carry
        lax.fori_loop(0, (n_chunks - 1) // 2, pbody, 0)

        wait_gathers(0)
        add_and_store(n_chunks - 1, 0)

    return pl.kernel(
        body,
        out_type=jax.ShapeDtypeStruct((n_edges, d), jnp.float32),
        mesh=_mesh(),
        scratch_types=[
            pltpu.VMEM((CHUNK,), jnp.int32),
            pltpu.VMEM((CHUNK,), jnp.int32),
            pltpu.VMEM((CHUNK,), jnp.int32),
            pltpu.VMEM((CHUNK,), jnp.int32),
            pltpu.VMEM((CHUNK, d), jnp.float32),
            pltpu.VMEM((CHUNK, d), jnp.float32),
            pltpu.VMEM((CHUNK, d), jnp.float32),
            pltpu.VMEM((CHUNK, d), jnp.float32),
            pltpu.SemaphoreType.DMA,
            pltpu.SemaphoreType.DMA,
            pltpu.SemaphoreType.DMA,
            pltpu.SemaphoreType.DMA,
        ],
    )


def _combine_body(ns_ref, deg_ref, out_ref):
    ns = ns_ref[0] + ns_ref[1]
    deg = deg_ref[0, :, 0] + deg_ref[1, :, 0]
    out_ref[...] = ns / jnp.maximum(deg, 1.0)[:, None]


def _linear_body(h_ref, w_ref, b_ref, out_ref):
    h = h_ref[0] + h_ref[1]
    p = lax.dot_general(h, w_ref[...], (((1,), (1,)), ((), ())),
                        preferred_element_type=jnp.float32)
    out_ref[...] = 0.5 * p + 0.5 * b_ref[...]


def kernel(inputs, edge_index, W, b):
    n_edges, d = inputs.shape
    n_nodes = ((_N_NODES + NS * CHUNK - 1) // (NS * CHUNK)) * (NS * CHUNK)
    src = edge_index[0]
    dst = edge_index[1]

    ns_p, deg_p = _seg_sum_call(n_nodes, n_edges, d)(inputs, dst)

    node_h = pl.pallas_call(
        _combine_body,
        out_shape=jax.ShapeDtypeStruct((n_nodes, d), jnp.float32),
    )(ns_p, deg_p)

    h_p = _gather_scatter_call(n_nodes, n_edges, d)(node_h, src, dst)

    p = pl.pallas_call(
        _linear_body,
        out_shape=jax.ShapeDtypeStruct((n_nodes, d), jnp.float32),
    )(h_p, W, b.reshape(1, d))

    return _edge_out_call(n_nodes, n_edges, d)(p, src, dst)


# speedup vs baseline: 6.3335x; 1.4751x over previous
import functools

import jax
import jax.numpy as jnp
from jax import lax
from jax.experimental import pallas as pl
from jax.experimental.pallas import tpu as pltpu
from jax.experimental.pallas import tpu_sc as plsc

_N_NODES = 10000

NC = 2
NS = 16
NW = NC * NS
LANES = 16
CHUNK = 80


def _mesh():
    return plsc.VectorSubcoreMesh(core_axis_name="c", subcore_axis_name="s")


def _zero_rows(rows_v, d):
    for r in range(CHUNK):
        for j in range(d // LANES):
            rows_v[r, pl.ds(j * LANES, LANES)] = jnp.zeros((LANES,), jnp.float32)


def _fill_iota(idx_v, start):
    for j in range(CHUNK // LANES):
        idx_v[pl.ds(j * LANES, LANES)] = (
            lax.iota(jnp.int32, LANES) + start + j * LANES)


def _seg_sum_call(n_nodes, n_edges, d):
    ew = n_edges // NW
    assert ew * NW == n_edges and ew % CHUNK == 0
    n_chunks = ew // CHUNK
    assert n_chunks % 2 == 1
    rpt = n_nodes // NS

    def body(inputs_hbm, dst_hbm, ns_out, deg_out,
             idx_v, i1a, i1b, ra, rb, ns_sh,
             sem, lsa, lsb, ssa, ssb):
        c = lax.axis_index("c")
        s = lax.axis_index("s")
        wid = s * NC + c
        r0 = s * rpt
        base = wid * ew
        idx1 = (i1a, i1b)
        rows2 = (ra, rb)
        lsem = (lsa, lsb)
        ssem = (ssa, ssb)

        _zero_rows(ra, d)
        for k in range(rpt // CHUNK):
            _fill_iota(idx_v, r0 + k * CHUNK)
            pltpu.sync_copy(ra, ns_sh.at[idx_v])
        plsc.subcore_barrier()

        def p1_loads(j, slot):
            off = base + j * CHUNK
            pltpu.async_copy(dst_hbm.at[pl.ds(off, CHUNK)], idx1[slot],
                             lsem[slot])
            pltpu.async_copy(inputs_hbm.at[pl.ds(off, CHUNK)], rows2[slot],
                             lsem[slot])

        def p1_wait_loads(slot):
            pltpu.make_async_copy(dst_hbm.at[pl.ds(base, CHUNK)], idx1[slot],
                                  lsem[slot]).wait()
            pltpu.make_async_copy(inputs_hbm.at[pl.ds(base, CHUNK)],
                                  rows2[slot], lsem[slot]).wait()

        def p1_scatter(slot):
            pltpu.async_copy(rows2[slot], ns_sh.at[idx1[slot]], ssem[slot],
                             add=True)

        def p1_wait_scatter(slot):
            pltpu.make_async_copy(rows2[slot], ns_sh.at[idx1[slot]],
                                  ssem[slot]).wait()

        p1_loads(0, 0)

        def p1_body(jp, carry):
            for b2 in range(2):
                j = jp * 2 + b2
                slot = b2
                other = 1 - b2
                p1_wait_loads(slot)
                p1_scatter(slot)

                @pl.when(j >= 1)
                def _():
                    p1_wait_scatter(other)
                p1_loads(j + 1, other)
            return carry
        lax.fori_loop(0, n_chunks // 2, p1_body, 0)
        p1_wait_loads(0)
        p1_scatter(0)
        p1_wait_scatter(1)
        p1_wait_scatter(0)
        plsc.subcore_barrier()

        for k in range(rpt // CHUNK):
            rk = r0 + k * CHUNK
            _fill_iota(idx_v, rk)
            pltpu.async_copy(ns_sh.at[idx_v], ra, sem).wait()
            pltpu.sync_copy(ra, ns_out.at[c, pl.ds(rk, CHUNK)])

        _zero_rows(ra, d)
        for k in range(rpt // CHUNK):
            _fill_iota(idx_v, r0 + k * CHUNK)
            pltpu.sync_copy(ra, ns_sh.at[idx_v])
        for r in range(CHUNK):
            for j in range(d // LANES):
                rb[r, pl.ds(j * LANES, LANES)] = jnp.ones((LANES,), jnp.float32)
        plsc.subcore_barrier()

        def p2_load(j, slot):
            pltpu.async_copy(dst_hbm.at[pl.ds(base + j * CHUNK, CHUNK)],
                             idx1[slot], lsem[slot])

        def p2_wait_load(slot):
            pltpu.make_async_copy(dst_hbm.at[pl.ds(base, CHUNK)], idx1[slot],
                                  lsem[slot]).wait()

        def p2_scatter(slot):
            pltpu.async_copy(rb, ns_sh.at[idx1[slot]], ssem[slot], add=True)

        def p2_wait_scatter(slot):
            pltpu.make_async_copy(rb, ns_sh.at[idx1[slot]], ssem[slot]).wait()

        p2_load(0, 0)

        def p2_body(jp, carry):
            for b2 in range(2):
                j = jp * 2 + b2
                slot = b2
                other = 1 - b2
                p2_wait_load(slot)
                p2_scatter(slot)

                @pl.when(j >= 1)
                def _():
                    p2_wait_scatter(other)
                p2_load(j + 1, other)
            return carry
        lax.fori_loop(0, n_chunks // 2, p2_body, 0)
        p2_wait_load(0)
        p2_scatter(0)
        p2_wait_scatter(1)
        p2_wait_scatter(0)
        plsc.subcore_barrier()

        for k in range(rpt // CHUNK):
            rk = r0 + k * CHUNK
            _fill_iota(idx_v, rk)
            pltpu.async_copy(ns_sh.at[idx_v], ra, sem).wait()
            pltpu.sync_copy(ra, deg_out.at[c, pl.ds(rk, CHUNK)])

    return pl.kernel(
        body,
        out_type=(
            jax.ShapeDtypeStruct((NC, n_nodes, d), jnp.float32),
            jax.ShapeDtypeStruct((NC, n_nodes, d), jnp.float32),
        ),
        mesh=_mesh(),
        scratch_types=[
            pltpu.VMEM((CHUNK,), jnp.int32),
            pltpu.VMEM((CHUNK,), jnp.int32),
            pltpu.VMEM((CHUNK,), jnp.int32),
            pltpu.VMEM((CHUNK, d), jnp.float32),
            pltpu.VMEM((CHUNK, d), jnp.float32),
            pltpu.VMEM_SHARED((n_nodes, d), jnp.float32),
            pltpu.SemaphoreType.DMA,
            pltpu.SemaphoreType.DMA,
            pltpu.SemaphoreType.DMA,
            pltpu.SemaphoreType.DMA,
            pltpu.SemaphoreType.DMA,
        ],
    )


def _gather_scatter_call(n_nodes, n_edges, d):
    ew = n_edges // NW
    n_chunks = ew // CHUNK
    assert n_chunks % 2 == 1
    rpt = n_nodes // NS

    def body(nodeh_hbm, src_hbm, dst_hbm, h_out,
             idx_v, sa, sb, da, db, ra, rb, h_sh,
             sem, isa_, isb_, gsa, gsb, ssa, ssb):
        c = lax.axis_index("c")
        s = lax.axis_index("s")
        wid = s * NC + c
        r0 = s * rpt
        base = wid * ew
        sidx = (sa, sb)
        didx = (da, db)
        rows2 = (ra, rb)
        isem = (isa_, isb_)
        gsem = (gsa, gsb)
        ssem = (ssa, ssb)

        _zero_rows(ra, d)
        for k in range(rpt // CHUNK):
            _fill_iota(idx_v, r0 + k * CHUNK)
            pltpu.sync_copy(ra, h_sh.at[idx_v])
        plsc.subcore_barrier()

        def c_idx(j, slot):
            off = base + j * CHUNK
            pltpu.async_copy(src_hbm.at[pl.ds(off, CHUNK)], sidx[slot],
                             isem[slot])
            pltpu.async_copy(dst_hbm.at[pl.ds(off, CHUNK)], didx[slot],
                             isem[slot])

        def c_wait_idx(slot):
            pltpu.make_async_copy(src_hbm.at[pl.ds(base, CHUNK)], sidx[slot],
                                  isem[slot]).wait()
            pltpu.make_async_copy(dst_hbm.at[pl.ds(base, CHUNK)], didx[slot],
                                  isem[slot]).wait()

        def c_gather(slot):
            pltpu.async_copy(nodeh_hbm.at[sidx[slot]], rows2[slot], gsem[slot])

        def c_wait_gather(slot):
            pltpu.make_async_copy(nodeh_hbm.at[pl.ds(0, CHUNK)], rows2[slot],
                                  gsem[slot]).wait()

        def c_scatter(slot):
            pltpu.async_copy(rows2[slot], h_sh.at[didx[slot]], ssem[slot],
                             add=True)

        def c_wait_scatter(slot):
            pltpu.make_async_copy(rows2[slot], h_sh.at[didx[slot]],
                                  ssem[slot]).wait()

        c_idx(0, 0)

        def cbody(jp, carry):
            for b2 in range(2):
                j = jp * 2 + b2
                slot = b2
                other = 1 - b2
                c_wait_idx(slot)
                c_gather(slot)

                @pl.when(j >= 1)
                def _():
                    c_wait_scatter(other)
                c_idx(j + 1, other)
                c_wait_gather(slot)
                c_scatter(slot)
            return carry
        lax.fori_loop(0, n_chunks // 2, cbody, 0)
        c_wait_idx(0)
        c_gather(0)
        c_wait_scatter(1)
        c_wait_gather(0)
        c_scatter(0)
        c_wait_scatter(0)
        plsc.subcore_barrier()

        for k in range(rpt // CHUNK):
            rk = r0 + k * CHUNK
            _fill_iota(idx_v, rk)
            pltpu.async_copy(h_sh.at[idx_v], ra, sem).wait()
            pltpu.sync_copy(ra, h_out.at[c, pl.ds(rk, CHUNK)])

    return pl.kernel(
        body,
        out_type=jax.ShapeDtypeStruct((NC, n_nodes, d), jnp.float32),
        mesh=_mesh(),
        scratch_types=[
            pltpu.VMEM((CHUNK,), jnp.int32),
            pltpu.VMEM((CHUNK,), jnp.int32),
            pltpu.VMEM((CHUNK,), jnp.int32),
            pltpu.VMEM((CHUNK,), jnp.int32),
            pltpu.VMEM((CHUNK,), jnp.int32),
            pltpu.VMEM((CHUNK, d), jnp.float32),
            pltpu.VMEM((CHUNK, d), jnp.float32),
            pltpu.VMEM_SHARED((n_nodes, d), jnp.float32),
            pltpu.SemaphoreType.DMA,
            pltpu.SemaphoreType.DMA,
            pltpu.SemaphoreType.DMA,
            pltpu.SemaphoreType.DMA,
            pltpu.SemaphoreType.DMA,
            pltpu.SemaphoreType.DMA,
            pltpu.SemaphoreType.DMA,
        ],
    )


def _edge_out_call(n_nodes, n_edges, d):
    ew = n_edges // NW
    n_chunks = ew // CHUNK
    assert n_chunks % 2 == 1

    def body(p_hbm, src_hbm, dst_hbm, out_hbm,
             si0, si1, di0, di1, av0, av1, bv0, bv1,
             gsem0, gsem1, isem0, isem1):
        c = lax.axis_index("c")
        s = lax.axis_index("s")
        wid = s * NC + c
        base = wid * ew
        limit = base + ew
        si = (si0, si1)
        di = (di0, di1)
        av = (av0, av1)
        bv = (bv0, bv1)
        gsem = (gsem0, gsem1)
        isem = (isem0, isem1)

        def start_idx(j, slot):
            off = base + j * CHUNK
            pltpu.async_copy(src_hbm.at[pl.ds(off, CHUNK)], si[slot], isem[slot])
            pltpu.async_copy(dst_hbm.at[pl.ds(off, CHUNK)], di[slot], isem[slot])

        def wait_idx(slot):
            pltpu.make_async_copy(src_hbm.at[pl.ds(base, CHUNK)], si[slot],
                                  isem[slot]).wait()
            pltpu.make_async_copy(dst_hbm.at[pl.ds(base, CHUNK)], di[slot],
                                  isem[slot]).wait()

        def start_gathers(slot):
            pltpu.async_copy(p_hbm.at[si[slot]], av[slot], gsem[slot])
            pltpu.async_copy(p_hbm.at[di[slot]], bv[slot], gsem[slot])

        def wait_gathers(slot):
            pltpu.make_async_copy(p_hbm.at[pl.ds(0, CHUNK)], av[slot],
                                  gsem[slot]).wait()
            pltpu.make_async_copy(p_hbm.at[pl.ds(0, CHUNK)], bv[slot],
                                  gsem[slot]).wait()

        def add_and_store(j, slot):
            def rbody(rr, rc):
                for q in range(8):
                    r = rr * 8 + q
                    for jj in range(d // LANES):
                        sl = pl.ds(jj * LANES, LANES)
                        av[slot][r, sl] = av[slot][r, sl] + bv[slot][r, sl]
                return rc
            lax.fori_loop(0, CHUNK // 8, rbody, 0)
            pltpu.sync_copy(av[slot], out_hbm.at[pl.ds(base + j * CHUNK, CHUNK)])

        start_idx(0, 0)
        wait_idx(0)
        start_gathers(0)
        start_idx(1, 1)

        def pbody(ip, carry):
            for b2 in range(2):
                j = ip * 2 + b2
                slot = b2
                other = 1 - b2
                wait_gathers(slot)
                wait_idx(other)
                start_gathers(other)

                @pl.when(base + (j + 2) * CHUNK < limit)
                def _():
                    start_idx(j + 2, slot)
                add_and_store(j, slot)
            return carry
        lax.fori_loop(0, (n_chunks - 1) // 2, pbody, 0)

        wait_gathers(0)
        add_and_store(n_chunks - 1, 0)

    return pl.kernel(
        body,
        out_type=jax.ShapeDtypeStruct((n_edges, d), jnp.float32),
        mesh=_mesh(),
        scratch_types=[
            pltpu.VMEM((CHUNK,), jnp.int32),
            pltpu.VMEM((CHUNK,), jnp.int32),
            pltpu.VMEM((CHUNK,), jnp.int32),
            pltpu.VMEM((CHUNK,), jnp.int32),
            pltpu.VMEM((CHUNK, d), jnp.float32),
            pltpu.VMEM((CHUNK, d), jnp.float32),
            pltpu.VMEM((CHUNK, d), jnp.float32),
            pltpu.VMEM((CHUNK, d), jnp.float32),
            pltpu.SemaphoreType.DMA,
            pltpu.SemaphoreType.DMA,
            pltpu.SemaphoreType.DMA,
            pltpu.SemaphoreType.DMA,
        ],
    )


def _combine_body(ns_ref, deg_ref, out_ref):
    ns = ns_ref[0] + ns_ref[1]
    deg = deg_ref[0, :, 0] + deg_ref[1, :, 0]
    out_ref[...] = ns / jnp.maximum(deg, 1.0)[:, None]


def _linear_body(h_ref, w_ref, b_ref, out_ref):
    h = h_ref[0] + h_ref[1]
    p = lax.dot_general(h, w_ref[...], (((1,), (1,)), ((), ())),
                        preferred_element_type=jnp.float32)
    out_ref[...] = 0.5 * p + 0.5 * b_ref[...]


def kernel(inputs, edge_index, W, b):
    n_edges, d = inputs.shape
    n_nodes = ((_N_NODES + NS * CHUNK - 1) // (NS * CHUNK)) * (NS * CHUNK)
    src = edge_index[0]
    dst = edge_index[1]

    ns_p, deg_p = _seg_sum_call(n_nodes, n_edges, d)(inputs, dst)

    node_h = pl.pallas_call(
        _combine_body,
        out_shape=jax.ShapeDtypeStruct((n_nodes, d), jnp.float32),
    )(ns_p, deg_p)

    h_p = _gather_scatter_call(n_nodes, n_edges, d)(node_h, src, dst)

    p = pl.pallas_call(
        _linear_body,
        out_shape=jax.ShapeDtypeStruct((n_nodes, d), jnp.float32),
    )(h_p, W, b.reshape(1, d))

    return _edge_out_call(n_nodes, n_edges, d)(p, src, dst)
